# baseline (speedup 1.0000x reference)
import functools

import jax
import jax.numpy as jnp
from jax import lax
from jax.experimental import pallas as pl
from jax.experimental.pallas import tpu as pltpu
from jax.experimental.pallas import tpu_sc as plsc

NN = 10000
EE = 320000
DD = 128
HH1 = 64
HH2 = 32

NPAD = 10112
NSUB = 16
CHK = 128
CHUNKS = 157
PER_SUB = CHUNKS * CHK
TOT_E = NSUB * PER_SUB
RPT = NPAD // 16
HALF = NPAD // 2
ACC_R = 5248
ARPT = ACC_R // 16

FROWS = 3840
URPT = FROWS // 32

BR = 2000
TR = 40


FW = 128


def _sc_degree_body(dst_hbm, out_hbm, dst_v, remap_v, ones_v, zero_v,
                    acc_sh):
    sid = lax.axis_index("s")
    cid = lax.axis_index("c")
    r0 = sid * ARPT

    def fillrow(i, carry):
        for k in range(FW // 16):
            ones_v[i, pl.ds(k * 16, 16)] = jnp.ones((16,), jnp.float32)
            zero_v[i, pl.ds(k * 16, 16)] = jnp.zeros((16,), jnp.float32)
        return carry

    lax.fori_loop(0, CHK, fillrow, 0)
    for k in range(2):
        pltpu.sync_copy(zero_v, acc_sh.at[pl.ds(r0 + k * CHK, CHK)])
    pltpu.sync_copy(zero_v.at[pl.ds(0, ARPT - 2 * CHK)],
                    acc_sh.at[pl.ds(r0 + 2 * CHK, ARPT - 2 * CHK)])
    pltpu.sync_copy(dst_hbm.at[sid], dst_v)
    plsc.subcore_barrier()

    base = cid * HALF

    def body(j, carry):
        for k in range(CHK // 16):
            d = dst_v[j, pl.ds(k * 16, 16)] - base
            m = (d >= 0) & (d < HALF)
            remap_v[pl.ds(k * 16, 16)] = jnp.where(m, d, HALF)
        pltpu.sync_copy(ones_v, acc_sh.at[remap_v], add=True)
        return carry

    lax.fori_loop(0, CHUNKS, body, 0)
    plsc.subcore_barrier()
    pltpu.sync_copy(acc_sh.at[pl.ds(r0, ARPT)], out_hbm.at[cid, pl.ds(r0, ARPT)])


def _sc_scatter_body(g_hbm, src_hbm, dst_hbm, out_hbm,
                     src_v, dst_v, remap_v, rows0_v, rows1_v,
                     acc_sh, gsem):
    cid = lax.axis_index("c")
    sid = lax.axis_index("s")
    r0 = sid * ARPT

    def zrow(i, carry):
        for k in range(FW // 16):
            rows0_v[i, pl.ds(k * 16, 16)] = jnp.zeros((16,), jnp.float32)
        return carry

    lax.fori_loop(0, CHK, zrow, 0)
    for k in range(2):
        pltpu.sync_copy(rows0_v, acc_sh.at[pl.ds(r0 + k * CHK, CHK)])
    pltpu.sync_copy(rows0_v.at[pl.ds(0, ARPT - 2 * CHK)],
                    acc_sh.at[pl.ds(r0 + 2 * CHK, ARPT - 2 * CHK)])

    pltpu.sync_copy(src_hbm.at[sid], src_v)
    pltpu.sync_copy(dst_hbm.at[sid], dst_v)
    plsc.subcore_barrier()

    base = cid * HALF

    def gfire(j, rref):
        pltpu.async_copy(g_hbm.at[src_v.at[j]], rref, gsem)

    def gwait(j, rref):
        pltpu.make_async_copy(g_hbm.at[src_v.at[j]], rref, gsem).wait()

    def scat(j, rref):
        for k in range(CHK // 16):
            d = dst_v[j, pl.ds(k * 16, 16)] - base
            m = (d >= 0) & (d < HALF)
            remap_v[pl.ds(k * 16, 16)] = jnp.where(m, d, HALF)
        pltpu.sync_copy(rref, acc_sh.at[remap_v], add=True)

    gfire(0, rows0_v)

    def body(i, carry):
        j0 = 2 * i
        j1 = 2 * i + 1

        @pl.when(j1 < CHUNKS)
        def _():
            gfire(j1, rows1_v)

        gwait(j0, rows0_v)
        scat(j0, rows0_v)

        @pl.when(j1 < CHUNKS)
        def _():
            @pl.when(j1 + 1 < CHUNKS)
            def _():
                gfire(j1 + 1, rows0_v)

            gwait(j1, rows1_v)
            scat(j1, rows1_v)

        return carry

    lax.fori_loop(0, (CHUNKS + 1) // 2, body, 0)
    plsc.subcore_barrier()
    pltpu.sync_copy(acc_sh.at[pl.ds(r0, ARPT)], out_hbm.at[cid, pl.ds(r0, ARPT)])


def _sc_u_body(z0_hbm, out_hbm, buf_v):
    del z0_hbm
    cid = lax.axis_index("c")
    sid = lax.axis_index("s")
    wid = cid * 16 + sid
    row0 = wid * URPT
    lane = lax.broadcasted_iota(jnp.uint32, (16,), 0)
    ks1 = jnp.full((16,), 1, jnp.uint32)
    ks2 = jnp.full((16,), 0x1BD11BDB, jnp.uint32)
    rot = ((13, 15, 26, 6), (17, 29, 16, 24))

    def blk(b, carry):
        base = (row0 + b * 8) * NN

        def vstep(v, carry2):
            t = jnp.full((16,), 1, jnp.uint32) * lax.convert_element_type(
                base + v * 16, jnp.uint32) + lane
            x0 = jnp.zeros((16,), jnp.uint32)
            x1 = t + ks1
            for i in range(5):
                for r in rot[i % 2]:
                    x0 = x0 + x1
                    x1 = (x1 << jnp.uint32(r)) | (x1 >> jnp.uint32(32 - r))
                    x1 = x1 ^ x0
                if (i + 1) % 3 == 1:
                    x0 = x0 + ks1
                elif (i + 1) % 3 == 2:
                    x0 = x0 + ks2
                if (i + 2) % 3 == 1:
                    x1 = x1 + ks1 + jnp.uint32(i + 1)
                elif (i + 2) % 3 == 2:
                    x1 = x1 + ks2 + jnp.uint32(i + 1)
                else:
                    x1 = x1 + jnp.uint32(i + 1)
            bits = x0 ^ x1
            r8 = v // 625
            c8 = v % 625
            buf_v[r8, pl.ds(c8 * 16, 16)] = bits
            return carry2

        lax.fori_loop(0, 8 * 625, vstep, 0)
        pltpu.sync_copy(buf_v, out_hbm.at[pl.ds(row0 + b * 8, 8)])
        return carry

    lax.fori_loop(0, URPT // 8, blk, 0)


@functools.cache
def _sc_kernels():
    mesh = plsc.VectorSubcoreMesh(core_axis_name="c", subcore_axis_name="s")
    degree = functools.partial(
        pl.kernel,
        mesh=mesh,
        out_type=jax.ShapeDtypeStruct((2, ACC_R, FW), jnp.float32),
        scratch_types=[
            pltpu.VMEM((CHUNKS, CHK), jnp.int32),
            pltpu.VMEM((CHK,), jnp.int32),
            pltpu.VMEM((CHK, FW), jnp.float32),
            pltpu.VMEM((CHK, FW), jnp.float32),
            pltpu.VMEM_SHARED((ACC_R, FW), jnp.float32),
        ],
    )(_sc_degree_body)
    scatter = functools.partial(
        pl.kernel,
        mesh=mesh,
        out_type=jax.ShapeDtypeStruct((2, ACC_R, FW), jnp.float32),
        scratch_types=[
            pltpu.VMEM((CHUNKS, CHK), jnp.int32),
            pltpu.VMEM((CHUNKS, CHK), jnp.int32),
            pltpu.VMEM((CHK,), jnp.int32),
            pltpu.VMEM((CHK, FW), jnp.float32),
            pltpu.VMEM((CHK, FW), jnp.float32),
            pltpu.VMEM_SHARED((ACC_R, FW), jnp.float32),
            pltpu.SemaphoreType.DMA,
        ],
    )(_sc_scatter_body)
    ukern = functools.partial(
        pl.kernel,
        mesh=mesh,
        out_type=jax.ShapeDtypeStruct((FROWS, NN), jnp.uint32),
        scratch_types=[
            pltpu.VMEM((8, NN), jnp.uint32),
        ],
    )(_sc_u_body)
    return degree, scatter, ukern


def _mm1_body(x_ref, w_ref, o_ref):
    o_ref[...] = jnp.dot(x_ref[...], w_ref[...],
                         preferred_element_type=jnp.float32)


def _g1_body(d_ref, xw_ref, g1_ref, dv_ref):
    deg = d_ref[:, 0:1] + 1.0
    dinv = 1.0 / jnp.sqrt(deg)
    g1_ref[...] = xw_ref[...] * dinv
    dv_ref[...] = jnp.broadcast_to(dinv, xw_ref.shape)


def _h_body(a_ref, g1_ref, dv_ref, b1_ref, w_ref, g2_ref):
    acc = a_ref[:, :HH1]
    h = jnp.maximum(
        dv_ref[...] * (acc + g1_ref[...]) + b1_ref[...],
        0.0)
    hw = jnp.dot(h, w_ref[...], preferred_element_type=jnp.float32)
    g2_ref[...] = hw * dv_ref[...]


def _z_body(c_ref, g2_ref, dv_ref, bc_ref, eps_ref,
            z_ref, mu_ref, lv_ref):
    acc = c_ref[:, :HH1]
    t = dv_ref[...] * (acc + g2_ref[...]) + bc_ref[...]
    mu = t[:, :HH2]
    logv = t[:, HH2:]
    mu_ref[...] = mu
    lv_ref[...] = logv
    z_ref[...] = mu + eps_ref[...] * jnp.exp(0.5 * logv)


def _dec_b_body(a_ref, zr_ref, zf_ref, u_ref, o_ref):
    del a_ref
    s = lax.dot_general(
        zr_ref[...], zf_ref[...], (((1,), (1,)), ((), ())),
        preferred_element_type=jnp.float32)
    a = jax.nn.sigmoid(s)
    fb = (u_ref[...] >> jnp.uint32(9)) | jnp.uint32(0x3F800000)
    u = lax.bitcast_convert_type(fb, jnp.float32) - 1.0
    o_ref[...] = (u < a).astype(jnp.float32)


def _dec_body(zr_ref, zf_ref, o_ref):
    p = pl.program_id(0) + FROWS // TR
    s = lax.dot_general(
        zr_ref[...], zf_ref[...], (((1,), (1,)), ((), ())),
        preferred_element_type=jnp.float32)
    a = jax.nn.sigmoid(s)
    r0 = lax.convert_element_type(p * TR, jnp.uint32)
    row = lax.broadcasted_iota(jnp.uint32, (TR, NN), 0)
    col = lax.broadcasted_iota(jnp.uint32, (TR, NN), 1)
    t = (row + r0) * jnp.uint32(NN) + col
    ks = (jnp.uint32(0), jnp.uint32(1), jnp.uint32(0x1BD11BDB))
    x0 = jnp.zeros_like(t) + ks[0]
    x1 = t + ks[1]
    rot = ((13, 15, 26, 6), (17, 29, 16, 24))
    for i in range(5):
        for r in rot[i % 2]:
            x0 = x0 + x1
            x1 = (x1 << jnp.uint32(r)) | (x1 >> jnp.uint32(32 - r))
            x1 = x1 ^ x0
        x0 = x0 + ks[(i + 1) % 3]
        x1 = x1 + ks[(i + 2) % 3] + jnp.uint32(i + 1)
    bits = x0 ^ x1
    fb = (bits >> jnp.uint32(9)) | jnp.uint32(0x3F800000)
    u = lax.bitcast_convert_type(fb, jnp.float32) - 1.0
    o_ref[...] = (u < a).astype(jnp.float32)


def _row_specs(widths):
    return [pl.BlockSpec((BR, w), lambda p: (p, 0)) for w in widths]


def kernel(x, edge_index, W1, b1, Wmu, bmu, Wlv, blv, eps):
    src = edge_index[0].astype(jnp.int32)
    dst = edge_index[1].astype(jnp.int32)
    pad = jnp.full((TOT_E - EE,), NN, jnp.int32)
    srcr = jnp.concatenate([src, pad]).reshape(NSUB, CHUNKS, CHK)
    dstr = jnp.concatenate([dst, pad]).reshape(NSUB, CHUNKS, CHK)

    sc_degree, sc_scatter, sc_u = _sc_kernels()

    degpp = sc_degree(dstr)
    degp = jnp.concatenate([degpp[0, :HALF], degpp[1, :HALF]], axis=0)

    xw1 = pl.pallas_call(
        _mm1_body,
        grid=(NN // BR,),
        in_specs=[pl.BlockSpec((BR, DD), lambda p: (p, 0)),
                  pl.BlockSpec((DD, HH1), lambda p: (0, 0))],
        out_specs=pl.BlockSpec((BR, HH1), lambda p: (p, 0)),
        out_shape=jax.ShapeDtypeStruct((NN, HH1), jnp.float32),
    )(x, W1)

    g1, dinv64 = pl.pallas_call(
        _g1_body,
        grid=(NN // BR,),
        in_specs=_row_specs([FW, HH1]),
        out_specs=_row_specs([HH1, HH1]),
        out_shape=[jax.ShapeDtypeStruct((NN, HH1), jnp.float32),
                   jax.ShapeDtypeStruct((NN, HH1), jnp.float32)],
    )(degp, xw1)

    g1p = jnp.pad(g1, ((0, NPAD - NN), (0, FW - HH1)))
    acc1p = sc_scatter(g1p, srcr, dstr)
    acc1 = jnp.concatenate([acc1p[0, :HALF], acc1p[1, :HALF]], axis=0)

    Wcat = jnp.concatenate([Wmu, Wlv], axis=1)
    bcat = jnp.concatenate([bmu, blv]).reshape(1, HH1)
    b1r = b1.reshape(1, HH1)

    g2 = pl.pallas_call(
        _h_body,
        grid=(NN // BR,),
        in_specs=_row_specs([FW, HH1, HH1]) + [
            pl.BlockSpec((1, HH1), lambda p: (0, 0)),
            pl.BlockSpec((HH1, HH1), lambda p: (0, 0))],
        out_specs=pl.BlockSpec((BR, HH1), lambda p: (p, 0)),
        out_shape=jax.ShapeDtypeStruct((NN, HH1), jnp.float32),
    )(acc1, g1, dinv64, b1r, Wcat)

    g2p = jnp.pad(g2, ((0, NPAD - NN), (0, FW - HH1)))
    acc2p = sc_scatter(g2p, srcr, dstr)
    acc2 = jnp.concatenate([acc2p[0, :HALF], acc2p[1, :HALF]], axis=0)

    z, mu, logvar = pl.pallas_call(
        _z_body,
        grid=(NN // BR,),
        in_specs=_row_specs([FW, HH1, HH1]) + [
            pl.BlockSpec((1, HH1), lambda p: (0, 0)),
            pl.BlockSpec((BR, HH2), lambda p: (p, 0))],
        out_specs=_row_specs([HH2, HH2, HH2]),
        out_shape=[jax.ShapeDtypeStruct((NN, HH2), jnp.float32),
                   jax.ShapeDtypeStruct((NN, HH2), jnp.float32),
                   jax.ShapeDtypeStruct((NN, HH2), jnp.float32)],
    )(acc2, g2, dinv64, bcat, eps)

    u_sc = sc_u(z)

    A_part = pl.pallas_call(
        _dec_body,
        grid=((NN - FROWS) // TR,),
        in_specs=[pl.BlockSpec((TR, HH2),
                               lambda p: (p + FROWS // TR, 0)),
                  pl.BlockSpec((NN, HH2), lambda p: (0, 0))],
        out_specs=pl.BlockSpec((TR, NN), lambda p: (p + FROWS // TR, 0)),
        out_shape=jax.ShapeDtypeStruct((NN, NN), jnp.float32),
    )(z, z)

    A_de = pl.pallas_call(
        _dec_b_body,
        grid=(FROWS // TR,),
        in_specs=[pl.BlockSpec((TR, NN), lambda p: (p, 0)),
                  pl.BlockSpec((TR, HH2), lambda p: (p, 0)),
                  pl.BlockSpec((NN, HH2), lambda p: (0, 0)),
                  pl.BlockSpec((TR, NN), lambda p: (p, 0))],
        out_specs=pl.BlockSpec((TR, NN), lambda p: (p, 0)),
        out_shape=jax.ShapeDtypeStruct((NN, NN), jnp.float32),
        input_output_aliases={0: 0},
    )(A_part, z, z, u_sc)

    return (z, A_de, mu, logvar)


# speedup vs baseline: 1.1896x; 1.1896x over previous
import functools

import jax
import jax.numpy as jnp
from jax import lax
from jax.experimental import pallas as pl
from jax.experimental.pallas import tpu as pltpu
from jax.experimental.pallas import tpu_sc as plsc

NN = 10000
EE = 320000
DD = 128
HH1 = 64
HH2 = 32

NPAD = 10112
NSUB = 16
CHK = 128
CHUNKS = 157
PER_SUB = CHUNKS * CHK
TOT_E = NSUB * PER_SUB
RPT = NPAD // 16
HALF = NPAD // 2
ACC_R = 5248
ARPT = ACC_R // 16

FROWS = 2560
URPT = FROWS // 32

BR = 2000
TR = 40


FW = 128


def _sc_degree_body(dst_hbm, out_hbm, dst_v, remap_v, ones_v, zero_v,
                    acc_sh):
    sid = lax.axis_index("s")
    cid = lax.axis_index("c")
    r0 = sid * ARPT

    def fillrow(i, carry):
        for k in range(FW // 16):
            ones_v[i, pl.ds(k * 16, 16)] = jnp.ones((16,), jnp.float32)
            zero_v[i, pl.ds(k * 16, 16)] = jnp.zeros((16,), jnp.float32)
        return carry

    lax.fori_loop(0, CHK, fillrow, 0)
    for k in range(2):
        pltpu.sync_copy(zero_v, acc_sh.at[pl.ds(r0 + k * CHK, CHK)])
    pltpu.sync_copy(zero_v.at[pl.ds(0, ARPT - 2 * CHK)],
                    acc_sh.at[pl.ds(r0 + 2 * CHK, ARPT - 2 * CHK)])
    pltpu.sync_copy(dst_hbm.at[sid], dst_v)
    plsc.subcore_barrier()

    base = cid * HALF

    def body(j, carry):
        for k in range(CHK // 16):
            d = dst_v[j, pl.ds(k * 16, 16)] - base
            m = (d >= 0) & (d < HALF)
            remap_v[pl.ds(k * 16, 16)] = jnp.where(m, d, HALF)
        pltpu.sync_copy(ones_v, acc_sh.at[remap_v], add=True)
        return carry

    lax.fori_loop(0, CHUNKS, body, 0)
    plsc.subcore_barrier()
    pltpu.sync_copy(acc_sh.at[pl.ds(r0, ARPT)], out_hbm.at[cid, pl.ds(r0, ARPT)])


def _sc_scatter_body(g_hbm, src_hbm, dst_hbm, out_hbm,
                     src_v, dst_v, remap_v, rows0_v, rows1_v,
                     acc_sh, gsem):
    cid = lax.axis_index("c")
    sid = lax.axis_index("s")
    r0 = sid * ARPT

    def zrow(i, carry):
        for k in range(FW // 16):
            rows0_v[i, pl.ds(k * 16, 16)] = jnp.zeros((16,), jnp.float32)
        return carry

    lax.fori_loop(0, CHK, zrow, 0)
    for k in range(2):
        pltpu.sync_copy(rows0_v, acc_sh.at[pl.ds(r0 + k * CHK, CHK)])
    pltpu.sync_copy(rows0_v.at[pl.ds(0, ARPT - 2 * CHK)],
                    acc_sh.at[pl.ds(r0 + 2 * CHK, ARPT - 2 * CHK)])

    pltpu.sync_copy(src_hbm.at[sid], src_v)
    pltpu.sync_copy(dst_hbm.at[sid], dst_v)
    plsc.subcore_barrier()

    base = cid * HALF

    def gfire(j, rref):
        pltpu.async_copy(g_hbm.at[src_v.at[j]], rref, gsem)

    def gwait(j, rref):
        pltpu.make_async_copy(g_hbm.at[src_v.at[j]], rref, gsem).wait()

    def scat(j, rref):
        for k in range(CHK // 16):
            d = dst_v[j, pl.ds(k * 16, 16)] - base
            m = (d >= 0) & (d < HALF)
            remap_v[pl.ds(k * 16, 16)] = jnp.where(m, d, HALF)
        pltpu.sync_copy(rref, acc_sh.at[remap_v], add=True)

    gfire(0, rows0_v)

    def body(i, carry):
        j0 = 2 * i
        j1 = 2 * i + 1

        @pl.when(j1 < CHUNKS)
        def _():
            gfire(j1, rows1_v)

        gwait(j0, rows0_v)
        scat(j0, rows0_v)

        @pl.when(j1 < CHUNKS)
        def _():
            @pl.when(j1 + 1 < CHUNKS)
            def _():
                gfire(j1 + 1, rows0_v)

            gwait(j1, rows1_v)
            scat(j1, rows1_v)

        return carry

    lax.fori_loop(0, (CHUNKS + 1) // 2, body, 0)
    plsc.subcore_barrier()
    pltpu.sync_copy(acc_sh.at[pl.ds(r0, ARPT)], out_hbm.at[cid, pl.ds(r0, ARPT)])


def _sc_u_body(z0_hbm, out_hbm, buf_v):
    del z0_hbm
    cid = lax.axis_index("c")
    sid = lax.axis_index("s")
    wid = cid * 16 + sid
    row0 = wid * URPT
    lane = lax.broadcasted_iota(jnp.uint32, (16,), 0)
    ks1 = jnp.full((16,), 1, jnp.uint32)
    ks2 = jnp.full((16,), 0x1BD11BDB, jnp.uint32)
    rot = ((13, 15, 26, 6), (17, 29, 16, 24))

    def blk(b, carry):
        base = (row0 + b * 8) * NN

        def vstep(v, carry2):
            t = jnp.full((16,), 1, jnp.uint32) * lax.convert_element_type(
                base + v * 16, jnp.uint32) + lane
            x0 = jnp.zeros((16,), jnp.uint32)
            x1 = t + ks1
            for i in range(5):
                for r in rot[i % 2]:
                    x0 = x0 + x1
                    x1 = (x1 << jnp.uint32(r)) | (x1 >> jnp.uint32(32 - r))
                    x1 = x1 ^ x0
                if (i + 1) % 3 == 1:
                    x0 = x0 + ks1
                elif (i + 1) % 3 == 2:
                    x0 = x0 + ks2
                if (i + 2) % 3 == 1:
                    x1 = x1 + ks1 + jnp.uint32(i + 1)
                elif (i + 2) % 3 == 2:
                    x1 = x1 + ks2 + jnp.uint32(i + 1)
                else:
                    x1 = x1 + jnp.uint32(i + 1)
            bits = x0 ^ x1
            r8 = v // 625
            c8 = v % 625
            buf_v[r8, pl.ds(c8 * 16, 16)] = bits
            return carry2

        lax.fori_loop(0, 8 * 625, vstep, 0)
        pltpu.sync_copy(buf_v, out_hbm.at[pl.ds(row0 + b * 8, 8)])
        return carry

    lax.fori_loop(0, URPT // 8, blk, 0)


@functools.cache
def _sc_kernels():
    mesh = plsc.VectorSubcoreMesh(core_axis_name="c", subcore_axis_name="s")
    degree = functools.partial(
        pl.kernel,
        mesh=mesh,
        out_type=jax.ShapeDtypeStruct((2, ACC_R, FW), jnp.float32),
        scratch_types=[
            pltpu.VMEM((CHUNKS, CHK), jnp.int32),
            pltpu.VMEM((CHK,), jnp.int32),
            pltpu.VMEM((CHK, FW), jnp.float32),
            pltpu.VMEM((CHK, FW), jnp.float32),
            pltpu.VMEM_SHARED((ACC_R, FW), jnp.float32),
        ],
    )(_sc_degree_body)
    scatter = functools.partial(
        pl.kernel,
        mesh=mesh,
        out_type=jax.ShapeDtypeStruct((2, ACC_R, FW), jnp.float32),
        scratch_types=[
            pltpu.VMEM((CHUNKS, CHK), jnp.int32),
            pltpu.VMEM((CHUNKS, CHK), jnp.int32),
            pltpu.VMEM((CHK,), jnp.int32),
            pltpu.VMEM((CHK, FW), jnp.float32),
            pltpu.VMEM((CHK, FW), jnp.float32),
            pltpu.VMEM_SHARED((ACC_R, FW), jnp.float32),
            pltpu.SemaphoreType.DMA,
        ],
    )(_sc_scatter_body)
    ukern = functools.partial(
        pl.kernel,
        mesh=mesh,
        out_type=jax.ShapeDtypeStruct((FROWS, NN), jnp.uint32),
        scratch_types=[
            pltpu.VMEM((8, NN), jnp.uint32),
        ],
    )(_sc_u_body)
    return degree, scatter, ukern


def _mm1_body(x_ref, w_ref, o_ref):
    o_ref[...] = jnp.dot(x_ref[...], w_ref[...],
                         preferred_element_type=jnp.float32)


def _g1_body(d_ref, xw_ref, g1_ref, dv_ref):
    deg = d_ref[:, 0:1] + 1.0
    dinv = 1.0 / jnp.sqrt(deg)
    g1_ref[...] = xw_ref[...] * dinv
    dv_ref[...] = jnp.broadcast_to(dinv, xw_ref.shape)


def _h_body(a_ref, g1_ref, dv_ref, b1_ref, w_ref, g2_ref):
    acc = a_ref[:, :HH1]
    h = jnp.maximum(
        dv_ref[...] * (acc + g1_ref[...]) + b1_ref[...],
        0.0)
    hw = jnp.dot(h, w_ref[...], preferred_element_type=jnp.float32)
    g2_ref[...] = hw * dv_ref[...]


def _z_body(c_ref, g2_ref, dv_ref, bc_ref, eps_ref,
            z_ref, mu_ref, lv_ref):
    acc = c_ref[:, :HH1]
    t = dv_ref[...] * (acc + g2_ref[...]) + bc_ref[...]
    mu = t[:, :HH2]
    logv = t[:, HH2:]
    mu_ref[...] = mu
    lv_ref[...] = logv
    z_ref[...] = mu + eps_ref[...] * jnp.exp(0.5 * logv)


def _dec_b_body(a_ref, zr_ref, zf_ref, u_ref, o_ref):
    del a_ref
    s = lax.dot_general(
        zr_ref[...], zf_ref[...], (((1,), (1,)), ((), ())),
        preferred_element_type=jnp.float32)
    a = jax.nn.sigmoid(s)
    fb = (u_ref[...] >> jnp.uint32(9)) | jnp.uint32(0x3F800000)
    u = lax.bitcast_convert_type(fb, jnp.float32) - 1.0
    o_ref[...] = (u < a).astype(jnp.float32)


def _dec_body(zr_ref, zf_ref, o_ref):
    p = pl.program_id(0) + FROWS // TR
    s = lax.dot_general(
        zr_ref[...], zf_ref[...], (((1,), (1,)), ((), ())),
        preferred_element_type=jnp.float32)
    a = jax.nn.sigmoid(s)
    r0 = lax.convert_element_type(p * TR, jnp.uint32)
    row = lax.broadcasted_iota(jnp.uint32, (TR, NN), 0)
    col = lax.broadcasted_iota(jnp.uint32, (TR, NN), 1)
    t = (row + r0) * jnp.uint32(NN) + col
    ks = (jnp.uint32(0), jnp.uint32(1), jnp.uint32(0x1BD11BDB))
    x0 = jnp.zeros_like(t) + ks[0]
    x1 = t + ks[1]
    rot = ((13, 15, 26, 6), (17, 29, 16, 24))
    for i in range(5):
        for r in rot[i % 2]:
            x0 = x0 + x1
            x1 = (x1 << jnp.uint32(r)) | (x1 >> jnp.uint32(32 - r))
            x1 = x1 ^ x0
        x0 = x0 + ks[(i + 1) % 3]
        x1 = x1 + ks[(i + 2) % 3] + jnp.uint32(i + 1)
    bits = x0 ^ x1
    fb = (bits >> jnp.uint32(9)) | jnp.uint32(0x3F800000)
    u = lax.bitcast_convert_type(fb, jnp.float32) - 1.0
    o_ref[...] = (u < a).astype(jnp.float32)


def _row_specs(widths):
    return [pl.BlockSpec((BR, w), lambda p: (p, 0)) for w in widths]


def kernel(x, edge_index, W1, b1, Wmu, bmu, Wlv, blv, eps):
    src = edge_index[0].astype(jnp.int32)
    dst = edge_index[1].astype(jnp.int32)
    pad = jnp.full((TOT_E - EE,), NN, jnp.int32)
    srcr = jnp.concatenate([src, pad]).reshape(NSUB, CHUNKS, CHK)
    dstr = jnp.concatenate([dst, pad]).reshape(NSUB, CHUNKS, CHK)

    sc_degree, sc_scatter, sc_u = _sc_kernels()

    degpp = sc_degree(dstr)
    degp = jnp.concatenate([degpp[0, :HALF], degpp[1, :HALF]], axis=0)

    xw1 = pl.pallas_call(
        _mm1_body,
        grid=(NN // BR,),
        in_specs=[pl.BlockSpec((BR, DD), lambda p: (p, 0)),
                  pl.BlockSpec((DD, HH1), lambda p: (0, 0))],
        out_specs=pl.BlockSpec((BR, HH1), lambda p: (p, 0)),
        out_shape=jax.ShapeDtypeStruct((NN, HH1), jnp.float32),
    )(x, W1)

    g1, dinv64 = pl.pallas_call(
        _g1_body,
        grid=(NN // BR,),
        in_specs=_row_specs([FW, HH1]),
        out_specs=_row_specs([HH1, HH1]),
        out_shape=[jax.ShapeDtypeStruct((NN, HH1), jnp.float32),
                   jax.ShapeDtypeStruct((NN, HH1), jnp.float32)],
    )(degp, xw1)

    g1p = jnp.pad(g1, ((0, NPAD - NN), (0, FW - HH1)))
    acc1p = sc_scatter(g1p, srcr, dstr)
    acc1 = jnp.concatenate([acc1p[0, :HALF], acc1p[1, :HALF]], axis=0)

    Wcat = jnp.concatenate([Wmu, Wlv], axis=1)
    bcat = jnp.concatenate([bmu, blv]).reshape(1, HH1)
    b1r = b1.reshape(1, HH1)

    g2 = pl.pallas_call(
        _h_body,
        grid=(NN // BR,),
        in_specs=_row_specs([FW, HH1, HH1]) + [
            pl.BlockSpec((1, HH1), lambda p: (0, 0)),
            pl.BlockSpec((HH1, HH1), lambda p: (0, 0))],
        out_specs=pl.BlockSpec((BR, HH1), lambda p: (p, 0)),
        out_shape=jax.ShapeDtypeStruct((NN, HH1), jnp.float32),
    )(acc1, g1, dinv64, b1r, Wcat)

    g2p = jnp.pad(g2, ((0, NPAD - NN), (0, FW - HH1)))
    acc2p = sc_scatter(g2p, srcr, dstr)
    acc2 = jnp.concatenate([acc2p[0, :HALF], acc2p[1, :HALF]], axis=0)

    z, mu, logvar = pl.pallas_call(
        _z_body,
        grid=(NN // BR,),
        in_specs=_row_specs([FW, HH1, HH1]) + [
            pl.BlockSpec((1, HH1), lambda p: (0, 0)),
            pl.BlockSpec((BR, HH2), lambda p: (p, 0))],
        out_specs=_row_specs([HH2, HH2, HH2]),
        out_shape=[jax.ShapeDtypeStruct((NN, HH2), jnp.float32),
                   jax.ShapeDtypeStruct((NN, HH2), jnp.float32),
                   jax.ShapeDtypeStruct((NN, HH2), jnp.float32)],
    )(acc2, g2, dinv64, bcat, eps)

    u_sc = sc_u(z)

    A_part = pl.pallas_call(
        _dec_body,
        grid=((NN - FROWS) // TR,),
        in_specs=[pl.BlockSpec((TR, HH2),
                               lambda p: (p + FROWS // TR, 0)),
                  pl.BlockSpec((NN, HH2), lambda p: (0, 0))],
        out_specs=pl.BlockSpec((TR, NN), lambda p: (p + FROWS // TR, 0)),
        out_shape=jax.ShapeDtypeStruct((NN, NN), jnp.float32),
    )(z, z)

    A_de = pl.pallas_call(
        _dec_b_body,
        grid=(FROWS // TR,),
        in_specs=[pl.BlockSpec((TR, NN), lambda p: (p, 0)),
                  pl.BlockSpec((TR, HH2), lambda p: (p, 0)),
                  pl.BlockSpec((NN, HH2), lambda p: (0, 0)),
                  pl.BlockSpec((TR, NN), lambda p: (p, 0))],
        out_specs=pl.BlockSpec((TR, NN), lambda p: (p, 0)),
        out_shape=jax.ShapeDtypeStruct((NN, NN), jnp.float32),
        input_output_aliases={0: 0},
    )(A_part, z, z, u_sc)

    return (z, A_de, mu, logvar)
